# baseline (speedup 1.0000x reference)
import functools

import jax
import jax.numpy as jnp
from jax.experimental import pallas as pl
from jax.experimental.pallas import tpu as pltpu
from jax.experimental.pallas import tpu_sc as plsc

EMB = 32
NODE_DIM = 4
NEG = -1e20
ROW = 2053
PROW = 2080
CHUNK = 8
NW = 32


def _prep_kernel(nfmt_ref, t1_ref, t2_ref, base_ref, m_ref):
    nfm_t = nfmt_ref[0]
    base = jax.lax.dot_general(
        nfm_t, t1_ref[...], (((0,), (0,)), ((), ())),
        preferred_element_type=jnp.float32)
    base_ref[0] = base
    mu1 = jnp.maximum(base, 0.0)
    m_ref[0] = jax.lax.dot_general(
        mu1, t2_ref[...], (((1,), (0,)), ((), ())),
        preferred_element_type=jnp.float32)


def _sc_pack_kernel(xf_hbm, wp_hbm, buf, obuf, *, rows_per_w):
    wid = jax.lax.axis_index("s") * 2 + jax.lax.axis_index("c")
    n_chunks = rows_per_w // CHUNK

    zeros16 = jnp.zeros((16,), jnp.uint32)
    buf[pl.ds(CHUNK * ROW, 16)] = zeros16
    buf[pl.ds(CHUNK * ROW + 16, 16)] = zeros16
    buf[pl.ds(CHUNK * ROW + 24, 16)] = zeros16

    @pl.loop(0, n_chunks)
    def _chunk(c):
        q = wid * n_chunks + c
        pltpu.sync_copy(xf_hbm.at[pl.ds(q * (CHUNK * ROW), CHUNK * ROW)],
                        buf.at[pl.ds(0, CHUNK * ROW)])
        for i in range(CHUNK):
            ib = i * ROW
            ob2 = i * (PROW // 2)
            for g in range(65):
                ua = buf[pl.ds(ib + 32 * g, 16)]
                ub = buf[pl.ds(ib + 32 * g + 16, 16)]
                ra = (ua + jnp.uint32(0x7FFF)
                      + ((ua >> jnp.uint32(16)) & jnp.uint32(1)))
                rb = (ub + jnp.uint32(0x7FFF)
                      + ((ub >> jnp.uint32(16)) & jnp.uint32(1)))
                obuf[pl.ds(ob2 + 16 * g, 16)] = (
                    (ra >> jnp.uint32(16)) | (rb & jnp.uint32(0xFFFF0000)))
        pltpu.sync_copy(obuf.at[pl.ds(0, CHUNK * PROW // 2)],
                        wp_hbm.at[pl.ds(q * (CHUNK * PROW // 2),
                                        CHUNK * PROW // 2)])


def _main_kernel(wp_ref, m_ref, base_ref, reach_ref, act_ref,
                 t3_ref, t4_ref, t5_ref, t5b_ref,
                 out_nl_ref, out_ap_ref,
                 s_scr, musum_scr, *, n_nodes, tile, k_steps):
    k = pl.program_id(1)

    @pl.when(k == 0)
    def _init():
        musum_scr[...] = jnp.zeros((1, EMB), jnp.float32)

    wm = jax.lax.dot_general(
        wp_ref[0], m_ref[0], (((1,), (0,)), ((), ())),
        preferred_element_type=jnp.float32)
    mu2 = jnp.maximum(base_ref[0] + wm, 0.0)
    musum_scr[...] += jnp.sum(mu2, axis=0, keepdims=True)
    loc = jnp.maximum(jax.lax.dot_general(
        mu2, t4_ref[...], (((1,), (0,)), ((), ())),
        preferred_element_type=jnp.float32), 0.0)
    s_row = jax.lax.dot_general(
        t5_ref[EMB:2 * EMB, 0:1], loc, (((0,), (1,)), ((), ())),
        preferred_element_type=jnp.float32)
    s_scr[:, pl.ds(k * tile, tile)] = s_row

    @pl.when(k == k_steps - 1)
    def _finish():
        g = jnp.maximum(jax.lax.dot_general(
            musum_scr[...], t3_ref[...], (((1,), (0,)), ((), ())),
            preferred_element_type=jnp.float32), 0.0)
        c = jax.lax.dot_general(
            g, t5_ref[0:EMB, 0:1], (((1,), (0,)), ((), ())),
            preferred_element_type=jnp.float32)[0, 0] + t5b_ref[0, 0]
        logits = s_scr[...] + c
        reach = reach_ref[0]
        logits = jnp.where(reach != 0.0, logits, NEG)
        mx = jnp.max(logits)
        lse = mx + jnp.log(jnp.sum(jnp.exp(logits - mx)))
        norm = logits - lse
        out_nl_ref[0] = norm
        a = act_ref[0, 0, 0]
        idx = jax.lax.broadcasted_iota(jnp.int32, (1, n_nodes), 1)
        out_ap_ref[0] = jnp.sum(jnp.where(idx == a, norm, 0.0),
                                axis=1, keepdims=True)


@jax.jit
def kernel(X, actions, theta1, theta2, theta3, theta4, theta5, theta5_b):
    if X.ndim == 2:
        X = X[None, ...]
    b_sz, n_nodes, row = X.shape
    tile = 512
    k_steps = n_nodes // tile
    total_rows = b_sz * n_nodes

    nfm_t = jnp.swapaxes(X[:, :, :NODE_DIM], 1, 2)
    reach = X[:, :, row - 1].reshape(b_sz, 1, n_nodes)
    acts = actions.astype(jnp.int32).reshape(b_sz, 1, 1)
    t5p = jnp.pad(theta5, ((0, 0), (0, 127)))
    t5b = theta5_b.reshape(1, 1)

    base, m = pl.pallas_call(
        _prep_kernel,
        grid=(b_sz,),
        in_specs=[
            pl.BlockSpec((1, NODE_DIM, n_nodes), lambda b: (b, 0, 0)),
            pl.BlockSpec((NODE_DIM, EMB), lambda b: (0, 0)),
            pl.BlockSpec((EMB, EMB), lambda b: (0, 0)),
        ],
        out_specs=[
            pl.BlockSpec((1, n_nodes, EMB), lambda b: (b, 0, 0)),
            pl.BlockSpec((1, n_nodes, EMB), lambda b: (b, 0, 0)),
        ],
        out_shape=[
            jax.ShapeDtypeStruct((b_sz, n_nodes, EMB), jnp.float32),
            jax.ShapeDtypeStruct((b_sz, n_nodes, EMB), jnp.float32),
        ],
    )(nfm_t, theta1, theta2)

    m_ext = jnp.zeros((b_sz, PROW, EMB), jnp.float32)
    m_ext = m_ext.at[:, NODE_DIM:NODE_DIM + n_nodes, :].set(m)
    m_perm = (m_ext.reshape(b_sz, PROW // 32, 2, 16, EMB)
              .swapaxes(2, 3)
              .reshape(b_sz, PROW, EMB)
              .astype(jnp.bfloat16))

    mesh = plsc.VectorSubcoreMesh(core_axis_name="c", subcore_axis_name="s")
    sc_pack = functools.partial(
        pl.kernel,
        mesh=mesh,
        out_type=jax.ShapeDtypeStruct((total_rows * PROW // 2,), jnp.uint32),
        scratch_types=[
            pltpu.VMEM((CHUNK * ROW + 40,), jnp.uint32),
            pltpu.VMEM((CHUNK * PROW // 2,), jnp.uint32),
        ],
    )(functools.partial(_sc_pack_kernel, rows_per_w=total_rows // NW))
    wp_u32 = sc_pack(jax.lax.bitcast_convert_type(X, jnp.uint32).reshape(total_rows * row))
    wp = jax.lax.bitcast_convert_type(
        wp_u32.reshape(b_sz, n_nodes, PROW // 2),
        jnp.bfloat16).reshape(b_sz, n_nodes, PROW)

    grid = (b_sz, k_steps)
    kern = functools.partial(_main_kernel, n_nodes=n_nodes, tile=tile,
                             k_steps=k_steps)
    norm_nl, act_p = pl.pallas_call(
        kern,
        grid=grid,
        in_specs=[
            pl.BlockSpec((1, tile, PROW), lambda b, k: (b, k, 0)),
            pl.BlockSpec((1, PROW, EMB), lambda b, k: (b, 0, 0)),
            pl.BlockSpec((1, tile, EMB), lambda b, k: (b, k, 0)),
            pl.BlockSpec((1, 1, n_nodes), lambda b, k: (b, 0, 0)),
            pl.BlockSpec((1, 1, 1), lambda b, k: (b, 0, 0)),
            pl.BlockSpec((EMB, EMB), lambda b, k: (0, 0)),
            pl.BlockSpec((EMB, EMB), lambda b, k: (0, 0)),
            pl.BlockSpec((2 * EMB, 128), lambda b, k: (0, 0)),
            pl.BlockSpec((1, 1), lambda b, k: (0, 0)),
        ],
        out_specs=[
            pl.BlockSpec((1, 1, n_nodes), lambda b, k: (b, 0, 0)),
            pl.BlockSpec((1, 1, 1), lambda b, k: (b, 0, 0)),
        ],
        out_shape=[
            jax.ShapeDtypeStruct((b_sz, 1, n_nodes), jnp.float32),
            jax.ShapeDtypeStruct((b_sz, 1, 1), jnp.float32),
        ],
        scratch_shapes=[
            pltpu.VMEM((1, n_nodes), jnp.float32),
            pltpu.VMEM((1, EMB), jnp.float32),
        ],
        compiler_params=pltpu.CompilerParams(
            dimension_semantics=("arbitrary", "arbitrary")),
    )(wp, m_perm, base, reach, acts, theta3, theta4, t5p, t5b)

    return norm_nl.reshape(b_sz, n_nodes), act_p.reshape(b_sz, 1)


# speedup vs baseline: 5.5735x; 5.5735x over previous
import functools

import jax
import jax.numpy as jnp
from jax.experimental import pallas as pl
from jax.experimental.pallas import tpu as pltpu

EMB = 32
NODE_DIM = 4
NEG = -1e20


def _fused_kernel(x_ref, nfmt_ref, reach_ref, act_ref, t1_ref, t2_ref,
                  t3_ref, t4_ref, t5_ref, t5b_ref,
                  out_nl_ref, out_ap_ref,
                  m_scr, base_scr, s_scr, musum_scr, *, n_nodes, tile, k_steps):
    k = pl.program_id(1)

    @pl.when(k == 0)
    def _init():
        nfm_t = nfmt_ref[0]
        base = jax.lax.dot_general(
            nfm_t, t1_ref[...], (((0,), (0,)), ((), ())),
            preferred_element_type=jnp.float32)
        base_scr[...] = base
        mu1 = jnp.maximum(base, 0.0)
        m = jax.lax.dot_general(
            mu1, t2_ref[...], (((1,), (0,)), ((), ())),
            preferred_element_type=jnp.float32)
        zpad = jnp.zeros((NODE_DIM, EMB), jnp.float32)
        m_scr[...] = jnp.concatenate([zpad, m, zpad],
                                     axis=0).astype(jnp.bfloat16)
        musum_scr[...] = jnp.zeros((1, EMB), jnp.float32)

    xt = x_ref[0]
    wm = jax.lax.dot_general(
        xt.astype(jnp.bfloat16), m_scr[0:n_nodes + NODE_DIM + 1, :],
        (((1,), (0,)), ((), ())),
        preferred_element_type=jnp.float32)
    base_t = base_scr[pl.ds(k * tile, tile), :]
    mu2 = jnp.maximum(base_t + wm, 0.0)
    musum_scr[...] += jnp.sum(mu2, axis=0, keepdims=True)
    loc = jnp.maximum(jax.lax.dot_general(
        mu2, t4_ref[...], (((1,), (0,)), ((), ())),
        preferred_element_type=jnp.float32), 0.0)
    s_row = jax.lax.dot_general(
        t5_ref[EMB:2 * EMB, 0:1], loc, (((0,), (1,)), ((), ())),
        preferred_element_type=jnp.float32)
    s_scr[:, pl.ds(k * tile, tile)] = s_row

    @pl.when(k == k_steps - 1)
    def _finish():
        g = jnp.maximum(jax.lax.dot_general(
            musum_scr[...], t3_ref[...], (((1,), (0,)), ((), ())),
            preferred_element_type=jnp.float32), 0.0)
        c = jax.lax.dot_general(
            g, t5_ref[0:EMB, 0:1], (((1,), (0,)), ((), ())),
            preferred_element_type=jnp.float32)[0, 0] + t5b_ref[0, 0]
        logits = s_scr[...] + c
        reach = reach_ref[0]
        logits = jnp.where(reach != 0.0, logits, NEG)
        mx = jnp.max(logits)
        lse = mx + jnp.log(jnp.sum(jnp.exp(logits - mx)))
        norm = logits - lse
        out_nl_ref[0] = norm
        a = act_ref[0, 0, 0]
        idx = jax.lax.broadcasted_iota(jnp.int32, (1, n_nodes), 1)
        out_ap_ref[0] = jnp.sum(jnp.where(idx == a, norm, 0.0),
                                axis=1, keepdims=True)


@jax.jit
def kernel(X, actions, theta1, theta2, theta3, theta4, theta5, theta5_b):
    if X.ndim == 2:
        X = X[None, ...]
    b_sz, n_nodes, row = X.shape
    tile = 512
    k_steps = n_nodes // tile

    nfm_t = jnp.swapaxes(X[:, :, :NODE_DIM], 1, 2)
    reach = X[:, :, row - 1].reshape(b_sz, 1, n_nodes)
    acts = actions.astype(jnp.int32).reshape(b_sz, 1, 1)
    t5p = jnp.pad(theta5, ((0, 0), (0, 127)))
    t5b = theta5_b.reshape(1, 1)

    grid = (b_sz, k_steps)
    kern = functools.partial(_fused_kernel, n_nodes=n_nodes, tile=tile,
                             k_steps=k_steps)
    norm_nl, act_p = pl.pallas_call(
        kern,
        grid=grid,
        in_specs=[
            pl.BlockSpec((1, tile, row), lambda b, k: (b, k, 0)),
            pl.BlockSpec((1, NODE_DIM, n_nodes), lambda b, k: (b, 0, 0)),
            pl.BlockSpec((1, 1, n_nodes), lambda b, k: (b, 0, 0)),
            pl.BlockSpec((1, 1, 1), lambda b, k: (b, 0, 0)),
            pl.BlockSpec((NODE_DIM, EMB), lambda b, k: (0, 0)),
            pl.BlockSpec((EMB, EMB), lambda b, k: (0, 0)),
            pl.BlockSpec((EMB, EMB), lambda b, k: (0, 0)),
            pl.BlockSpec((EMB, EMB), lambda b, k: (0, 0)),
            pl.BlockSpec((2 * EMB, 128), lambda b, k: (0, 0)),
            pl.BlockSpec((1, 1), lambda b, k: (0, 0)),
        ],
        out_specs=[
            pl.BlockSpec((1, 1, n_nodes), lambda b, k: (b, 0, 0)),
            pl.BlockSpec((1, 1, 1), lambda b, k: (b, 0, 0)),
        ],
        out_shape=[
            jax.ShapeDtypeStruct((b_sz, 1, n_nodes), jnp.float32),
            jax.ShapeDtypeStruct((b_sz, 1, 1), jnp.float32),
        ],
        scratch_shapes=[
            pltpu.VMEM((n_nodes + 2 * NODE_DIM, EMB), jnp.bfloat16),
            pltpu.VMEM((n_nodes, EMB), jnp.float32),
            pltpu.VMEM((1, n_nodes), jnp.float32),
            pltpu.VMEM((1, EMB), jnp.float32),
        ],
        compiler_params=pltpu.CompilerParams(
            dimension_semantics=("arbitrary", "arbitrary")),
    )(X, nfm_t, reach, acts, theta1, theta2, theta3, theta4, t5p, t5b)

    return norm_nl.reshape(b_sz, n_nodes), act_p.reshape(b_sz, 1)
